# baseline (device time: 13396 ns/iter reference)
import jax
import jax.numpy as jnp
from jax import lax
from jax.experimental import pallas as pl
from jax.experimental.pallas import tpu as pltpu

N_Y = 4


def kernel(x):
    m, n = x.shape
    blk = m
    assert n == N_Y * blk

    def body(x_ref, out_ref, x_vmem, send_buf, copy_sems, send_sems, recv_sems):
        my_x = lax.axis_index("x")
        my_y = lax.axis_index("y")
        my_z = lax.axis_index("z")

        barrier_sem = pltpu.get_barrier_semaphore()

        for i in range(N_Y):

            @pl.when(my_y == i)
            def _(i=i):
                near_first = sorted(
                    (j for j in range(N_Y) if j != i),
                    key=lambda j: abs(j - i),
                )

                for j in near_first:
                    pl.semaphore_signal(
                        barrier_sem,
                        inc=1,
                        device_id=(my_x, j, my_z),
                        device_id_type=pl.DeviceIdType.MESH,
                    )

                order = near_first + [i]
                for j in order:
                    pltpu.make_async_copy(
                        x_ref.at[:, pl.ds(j * blk, blk)],
                        x_vmem.at[j],
                        copy_sems.at[j],
                    ).start()

                for j in near_first:
                    pltpu.make_async_copy(
                        x_ref.at[:, pl.ds(j * blk, blk)],
                        x_vmem.at[j],
                        copy_sems.at[j],
                    ).wait()
                    send_buf[j] = x_vmem[j].astype(jnp.bfloat16)

                pl.semaphore_wait(barrier_sem, N_Y - 1)

                for j in near_first:
                    pltpu.make_async_remote_copy(
                        src_ref=send_buf.at[j],
                        dst_ref=out_ref.at[pl.ds(i * blk, blk)],
                        send_sem=send_sems.at[j],
                        recv_sem=recv_sems.at[i],
                        device_id=(my_x, j, my_z),
                        device_id_type=pl.DeviceIdType.MESH,
                    ).start()

                pltpu.make_async_copy(
                    x_ref.at[:, pl.ds(i * blk, blk)],
                    x_vmem.at[i],
                    copy_sems.at[i],
                ).wait()
                send_buf[i] = x_vmem[i].astype(jnp.bfloat16)
                pltpu.make_async_copy(
                    send_buf.at[i],
                    out_ref.at[pl.ds(i * blk, blk)],
                    copy_sems.at[i],
                ).start()

                for s in near_first:
                    pltpu.make_async_remote_copy(
                        src_ref=send_buf.at[s],
                        dst_ref=out_ref.at[pl.ds(s * blk, blk)],
                        send_sem=send_sems.at[s],
                        recv_sem=recv_sems.at[s],
                        device_id=(my_x, s, my_z),
                        device_id_type=pl.DeviceIdType.MESH,
                    ).wait_recv()

                pltpu.make_async_copy(
                    send_buf.at[i],
                    out_ref.at[pl.ds(i * blk, blk)],
                    copy_sems.at[i],
                ).wait()
                for j in near_first:
                    pltpu.make_async_remote_copy(
                        src_ref=send_buf.at[j],
                        dst_ref=out_ref.at[pl.ds(i * blk, blk)],
                        send_sem=send_sems.at[j],
                        recv_sem=recv_sems.at[i],
                        device_id=(my_x, j, my_z),
                        device_id_type=pl.DeviceIdType.MESH,
                    ).wait_send()

    return pl.pallas_call(
        body,
        out_shape=jax.ShapeDtypeStruct((N_Y * blk, blk), jnp.bfloat16),
        in_specs=[pl.BlockSpec(memory_space=pl.ANY)],
        out_specs=pl.BlockSpec(memory_space=pl.ANY),
        scratch_shapes=[
            pltpu.VMEM((N_Y, blk, blk), x.dtype),
            pltpu.VMEM((N_Y, blk, blk), jnp.bfloat16),
            pltpu.SemaphoreType.DMA((N_Y,)),
            pltpu.SemaphoreType.DMA((N_Y,)),
            pltpu.SemaphoreType.DMA((N_Y,)),
        ],
        compiler_params=pltpu.CompilerParams(collective_id=0),
    )(x)


# device time: 12234 ns/iter; 1.0950x vs baseline; 1.0950x over previous
import jax
import jax.numpy as jnp
from jax import lax
from jax.experimental import pallas as pl
from jax.experimental.pallas import tpu as pltpu

N_Y = 4


def kernel(x):
    m, n = x.shape
    blk = m
    assert n == N_Y * blk

    def body(x_ref, out_ref, x_vmem, send_buf, copy_sems, send_sems, recv_sems):
        my_x = lax.axis_index("x")
        my_y = lax.axis_index("y")
        my_z = lax.axis_index("z")

        barrier_sem = pltpu.get_barrier_semaphore()

        for i in range(N_Y):

            @pl.when(my_y == i)
            def _(i=i):
                near_first = sorted(
                    (j for j in range(N_Y) if j != i),
                    key=lambda j: abs(j - i),
                )

                for j in near_first:
                    pl.semaphore_signal(
                        barrier_sem,
                        inc=1,
                        device_id=(my_x, j, my_z),
                        device_id_type=pl.DeviceIdType.MESH,
                    )

                order = near_first + [i]
                for j in order:
                    pltpu.make_async_copy(
                        x_ref.at[:, pl.ds(j * blk, blk)],
                        x_vmem.at[j],
                        copy_sems.at[j],
                    ).start()

                for j in near_first:
                    pltpu.make_async_copy(
                        x_ref.at[:, pl.ds(j * blk, blk)],
                        x_vmem.at[j],
                        copy_sems.at[j],
                    ).wait()
                    send_buf[j] = x_vmem[j].astype(jnp.bfloat16)

                pl.semaphore_wait(barrier_sem, N_Y - 1)

                for j in near_first:
                    pltpu.make_async_remote_copy(
                        src_ref=send_buf.at[j],
                        dst_ref=out_ref.at[pl.ds(i * blk, blk)],
                        send_sem=send_sems.at[j],
                        recv_sem=recv_sems.at[i],
                        device_id=(my_x, j, my_z),
                        device_id_type=pl.DeviceIdType.MESH,
                    ).start()

                pltpu.make_async_copy(
                    x_ref.at[:, pl.ds(i * blk, blk)],
                    x_vmem.at[i],
                    copy_sems.at[i],
                ).wait()
                send_buf[i] = x_vmem[i].astype(jnp.bfloat16)
                pltpu.make_async_copy(
                    send_buf.at[i],
                    out_ref.at[pl.ds(i * blk, blk)],
                    copy_sems.at[i],
                ).start()

                for s in near_first:
                    pltpu.make_async_remote_copy(
                        src_ref=send_buf.at[s],
                        dst_ref=out_ref.at[pl.ds(s * blk, blk)],
                        send_sem=send_sems.at[s],
                        recv_sem=recv_sems.at[s],
                        device_id=(my_x, s, my_z),
                        device_id_type=pl.DeviceIdType.MESH,
                    ).wait_recv()

                pltpu.make_async_copy(
                    send_buf.at[i],
                    out_ref.at[pl.ds(i * blk, blk)],
                    copy_sems.at[i],
                ).wait()
                for j in near_first:
                    pltpu.make_async_remote_copy(
                        src_ref=send_buf.at[j],
                        dst_ref=out_ref.at[pl.ds(i * blk, blk)],
                        send_sem=send_sems.at[j],
                        recv_sem=recv_sems.at[i],
                        device_id=(my_x, j, my_z),
                        device_id_type=pl.DeviceIdType.MESH,
                    ).wait_send()

    return pl.pallas_call(
        body,
        out_shape=jax.ShapeDtypeStruct((N_Y * blk, blk), jnp.bfloat16),
        in_specs=[pl.BlockSpec(memory_space=pltpu.MemorySpace.HBM)],
        out_specs=pl.BlockSpec(memory_space=pltpu.MemorySpace.HBM),
        scratch_shapes=[
            pltpu.VMEM((N_Y, blk, blk), x.dtype),
            pltpu.VMEM((N_Y, blk, blk), jnp.bfloat16),
            pltpu.SemaphoreType.DMA((N_Y,)),
            pltpu.SemaphoreType.DMA((N_Y,)),
            pltpu.SemaphoreType.DMA((N_Y,)),
        ],
        compiler_params=pltpu.CompilerParams(collective_id=0),
    )(x)
